# baseline (speedup 1.0000x reference)
import jax
import jax.numpy as jnp
from jax.experimental import pallas as pl
from jax.experimental.pallas import tpu as pltpu

_S = 128
_CW = 512
_FR = 128
_KW = 576
_OFF = 225


def _pqmf_mm_kernel(xf_ref, hp_ref, hn_ref, v0_ref, v1_ref, o_ref):
    i = pl.program_id(1)
    n_t = pl.num_programs(1)
    for bl in range(xf_ref.shape[0]):
        F = xf_ref[bl].astype(jnp.bfloat16)
        prev6 = jnp.where(i == 0, 0.0,
                          hp_ref[bl, 2:8, :]).astype(jnp.bfloat16)
        next2 = jnp.where(i == n_t - 1, 0.0,
                          hn_ref[bl, 0:2, :]).astype(jnp.bfloat16)
        fall = jnp.concatenate([prev6, F, next2], axis=0)
        xf2 = fall.reshape(fall.shape[0] // 4, 4 * _FR)
        gt2 = xf2.shape[0] - 2
        Y0 = xf2[1:1 + gt2]
        Y1 = xf2[2:2 + gt2]
        R = (jnp.dot(Y0, v0_ref[...], preferred_element_type=jnp.float32)
             + jnp.dot(Y1, v1_ref[...], preferred_element_type=jnp.float32))
        o_ref[bl] = R.reshape(4 * gt2, _FR)


def kernel(x, H):
    B, Cc, T = x.shape
    Nb, taps1 = H.shape
    Tq = T // Nb
    pad = (taps1 - 1) // 2

    x2 = x.reshape(B, T)
    if T % _CW:
        x2 = jnp.pad(x2, ((0, 0), (0, _CW - T % _CW)))
    Tp = x2.shape[1]
    G = Tp // 256
    xf = x2.reshape(B, 2 * G, _FR)

    Gt = G
    for cand in (1024, 512, 256, 128, 64, 32, 16, 8):
        if G % cand == 0:
            Gt = cand
            break
    n_t = G // Gt

    per = 2 * _CW + 4
    u = jnp.pad(H.astype(jnp.float32), ((0, 0), (0, per - taps1)))
    reps = (per - _OFF + _S * 2 * _CW + per - 1) // per
    Vtall = (jnp.tile(u, (1, reps))[:, per - _OFF:per - _OFF + _S * 2 * _CW]
                .reshape(Nb * _S, 2 * _CW)
                .T.astype(jnp.bfloat16))

    flops = 2 * B * (G // 2) * 2 * _CW * (Nb * _S)
    bytes_accessed = 4 * B * G * 256 + 4 * B * Nb * G * 64

    bb = 1
    for cand_b in (4, 2):
        if B % cand_b == 0:
            bb = cand_b
            break
    fb = 2 * Gt // 8
    out = pl.pallas_call(
        _pqmf_mm_kernel,
        out_shape=jax.ShapeDtypeStruct((B, 2 * G, _FR), jnp.float32),
        grid=(B // bb, n_t),
        in_specs=[
            pl.BlockSpec((bb, 2 * Gt, _FR), lambda b, i: (b, i, 0)),
            pl.BlockSpec((bb, 8, _FR),
                         lambda b, i: (b, jnp.maximum(i * fb - 1, 0), 0)),
            pl.BlockSpec((bb, 8, _FR),
                         lambda b, i: (b, jnp.minimum((i + 1) * fb,
                                                      2 * G // 8 - 1), 0)),
            pl.BlockSpec((_CW, Nb * _S), lambda b, i: (0, 0)),
            pl.BlockSpec((_CW, Nb * _S), lambda b, i: (1, 0)),
        ],
        out_specs=pl.BlockSpec((bb, 2 * Gt, _FR), lambda b, i: (b, i, 0)),
        compiler_params=pltpu.CompilerParams(
            dimension_semantics=("parallel", "parallel")),
        cost_estimate=pl.CostEstimate(flops=int(flops), transcendentals=0,
                                      bytes_accessed=int(bytes_accessed)),
    )(xf, xf, xf, Vtall, Vtall)

    res = (out.reshape(B, G // 2, Nb, _S)
              .transpose(0, 2, 1, 3)
              .reshape(B, Nb, (G // 2) * _S))
    return res[:, :, :Tq]


# speedup vs baseline: 1.0477x; 1.0477x over previous
import jax
import jax.numpy as jnp
from jax.experimental import pallas as pl
from jax.experimental.pallas import tpu as pltpu

_S = 128
_CW = 512
_FR = 128
_KW = 576
_OFF = 225


def _pqmf_mm_kernel(xf_ref, hp_ref, hn_ref, v0_ref, v1_ref, o_ref):
    i = pl.program_id(1)
    n_t = pl.num_programs(1)
    for bl in range(xf_ref.shape[0]):
        F = xf_ref[bl].astype(jnp.bfloat16)
        prev6 = jnp.where(i == 0, 0.0,
                          hp_ref[bl, 2:8, :]).astype(jnp.bfloat16)
        next2 = jnp.where(i == n_t - 1, 0.0,
                          hn_ref[bl, 0:2, :]).astype(jnp.bfloat16)
        fall = jnp.concatenate([prev6, F, next2], axis=0)
        xf2 = fall.reshape(fall.shape[0] // 4, 4 * _FR)
        gt2 = xf2.shape[0] - 2
        Y0 = xf2[1:1 + gt2]
        Y1 = xf2[2:2 + gt2]
        R = (jnp.dot(Y0, v0_ref[...], preferred_element_type=jnp.float32)
             + jnp.dot(Y1, v1_ref[...], preferred_element_type=jnp.float32))
        o_ref[bl] = R.reshape(4 * gt2, _FR)


def kernel(x, H):
    B, Cc, T = x.shape
    Nb, taps1 = H.shape
    Tq = T // Nb
    pad = (taps1 - 1) // 2

    x2 = x.reshape(B, T)
    if T % _CW:
        x2 = jnp.pad(x2, ((0, 0), (0, _CW - T % _CW)))
    Tp = x2.shape[1]
    G = Tp // 256
    xf = x2.reshape(B, 2 * G, _FR)

    Gt = G
    for cand in (1024, 512, 256, 128, 64, 32, 16, 8):
        if G % cand == 0:
            Gt = cand
            break
    n_t = G // Gt

    u = jnp.pad(H.astype(jnp.float32), ((0, 0), (0, _KW + 4 - taps1)))
    flat = jnp.tile(u, (1, _S))[:, :_S * _KW]
    W2 = (flat.reshape(Nb, _S, _KW)
              .transpose(2, 0, 1)
              .reshape(_KW, Nb * _S))
    V0 = jnp.concatenate(
        [jnp.zeros((_OFF, Nb * _S), jnp.float32), W2[:_CW - _OFF]],
        axis=0).astype(jnp.bfloat16)
    V1 = jnp.concatenate(
        [W2[_CW - _OFF:], jnp.zeros((2 * _CW - _KW - _OFF, Nb * _S),
                                    jnp.float32)], axis=0).astype(jnp.bfloat16)

    flops = 2 * B * (G // 2) * 2 * _CW * (Nb * _S)
    bytes_accessed = 4 * B * G * 256 + 4 * B * Nb * G * 64

    bb = 1
    for cand_b in (4, 2):
        if B % cand_b == 0:
            bb = cand_b
            break
    fb = 2 * Gt // 8
    out = pl.pallas_call(
        _pqmf_mm_kernel,
        out_shape=jax.ShapeDtypeStruct((B, 2 * G, _FR), jnp.float32),
        grid=(B // bb, n_t),
        in_specs=[
            pl.BlockSpec((bb, 2 * Gt, _FR), lambda b, i: (b, i, 0)),
            pl.BlockSpec((bb, 8, _FR),
                         lambda b, i: (b, jnp.maximum(i * fb - 1, 0), 0)),
            pl.BlockSpec((bb, 8, _FR),
                         lambda b, i: (b, jnp.minimum((i + 1) * fb,
                                                      2 * G // 8 - 1), 0)),
            pl.BlockSpec((_CW, Nb * _S), lambda b, i: (0, 0)),
            pl.BlockSpec((_CW, Nb * _S), lambda b, i: (0, 0)),
        ],
        out_specs=pl.BlockSpec((bb, 2 * Gt, _FR), lambda b, i: (b, i, 0)),
        compiler_params=pltpu.CompilerParams(
            dimension_semantics=("parallel", "parallel")),
        cost_estimate=pl.CostEstimate(flops=int(flops), transcendentals=0,
                                      bytes_accessed=int(bytes_accessed)),
    )(xf, xf, xf, V0, V1)

    res = (out.reshape(B, G // 2, Nb, _S)
              .transpose(0, 2, 1, 3)
              .reshape(B, Nb, (G // 2) * _S))
    return res[:, :, :Tq]
